# baseline (speedup 1.0000x reference)
import functools
import jax
import jax.numpy as jnp
from jax import lax
from jax.experimental import pallas as pl
from jax.experimental.pallas import tpu as pltpu
from jax.experimental.pallas import tpu_sc as plsc

EMB = 768
HEADS = 12
BUCKET = 64
NHASH = 8
T = 4096
DH = EMB // HEADS
NB = T // BUCKET
NCH = NHASH * T // BUCKET
CB = 16
NR = NHASH * T
NTASK = HEADS * NHASH
OW = 128


def _proj_hash_body(x_ref, wqk_ref, wv_ref, rot_ref, qkv_ref, b_ref):
    qk = jnp.dot(x_ref[...], wqk_ref[0], preferred_element_type=jnp.float32)
    v = jnp.dot(x_ref[...], wv_ref[0], preferred_element_type=jnp.float32)
    qkv_ref[0, :, 0:DH] = qk
    qkv_ref[0, :, DH:2 * DH] = v
    rT = lax.dot_general(rot_ref[...], qk, (((1,), (1,)), ((), ())),
                         preferred_element_type=jnp.float32)
    half = NB // 2
    sub_iota = lax.broadcasted_iota(jnp.int32, (NB, T), 0)
    for r in range(NHASH):
        rr = rT[half * r:half * (r + 1), :]
        full = jnp.concatenate([rr, -rr], axis=0)
        mx = jnp.max(full, axis=0, keepdims=True)
        idx = jnp.min(jnp.where(full == mx, sub_iota, NB), axis=0, keepdims=True)
        b_ref[0, r:r + 1, :] = idx


def _proj_hash(x2, Wqk3, Wv3, rotT):
    return pl.pallas_call(
        _proj_hash_body,
        grid=(HEADS,),
        in_specs=[
            pl.BlockSpec((T, EMB), lambda h: (0, 0)),
            pl.BlockSpec((1, EMB, DH), lambda h: (h, 0, 0)),
            pl.BlockSpec((1, EMB, DH), lambda h: (h, 0, 0)),
            pl.BlockSpec((NHASH * (NB // 2), DH), lambda h: (0, 0)),
        ],
        out_specs=[
            pl.BlockSpec((1, T, 2 * DH), lambda h: (h, 0, 0)),
            pl.BlockSpec((1, NHASH, T), lambda h: (h, 0, 0)),
        ],
        out_shape=[
            jax.ShapeDtypeStruct((HEADS, T, 2 * DH), jnp.float32),
            jax.ShapeDtypeStruct((HEADS, NHASH, T), jnp.int32),
        ],
    )(x2, Wqk3, Wv3, rotT)


def _dst_body(b_ref, dst_ref, bm_ref, dloc_ref, ot_ref):
    h = pl.program_id(0)
    C = 128
    NCHK = T // C
    su = (lax.broadcasted_iota(jnp.int32, (C, C), 0)
          < lax.broadcasted_iota(jnp.int32, (C, C), 1)).astype(jnp.bfloat16)
    sl = (lax.broadcasted_iota(jnp.int32, (NB, NB), 1)
          < lax.broadcasted_iota(jnp.int32, (NB, NB), 0)).astype(jnp.float32)
    iota_col = lax.broadcasted_iota(jnp.int32, (NB, C), 0)

    for r in range(NHASH):
        hist = jnp.zeros((NB, 1), jnp.float32)
        for i in range(NCHK):
            b_row = b_ref[0, r:r + 1, pl.ds(i * C, C)]
            otb = (iota_col == b_row).astype(jnp.bfloat16)
            ot_ref[:, pl.ds(i * C, C)] = otb
            hist = hist + jnp.sum(otb, axis=1, keepdims=True
                                  ).astype(jnp.float32)
        offs = jnp.dot(sl, hist, preferred_element_type=jnp.float32)
        goff = (h * NHASH + r) * T

        base = jnp.zeros((NB, 1), jnp.float32)
        for i in range(NCHK):
            otb = ot_ref[:, pl.ds(i * C, C)]
            rank = jnp.dot(otb, su, preferred_element_type=jnp.float32)
            val = (rank + base + offs) * otb.astype(jnp.float32)
            dstv = jnp.sum(val, axis=0, keepdims=True)
            dloc_ref[r:r + 1, pl.ds(i * C, C)] = dstv.astype(jnp.int32)
            dst_ref[0, r:r + 1, pl.ds(i * C, C)] = dstv.astype(jnp.int32) + goff
            base = base + jnp.sum(otb, axis=1, keepdims=True
                                  ).astype(jnp.float32)

    iota_bk = lax.broadcasted_iota(jnp.int32, (BUCKET, T), 0)
    for r in range(NHASH):
        rp = (r - 1) % NHASH
        at = (iota_bk == dloc_ref[r:r + 1, :]).astype(jnp.bfloat16)
        bt = ((iota_bk + (T - BUCKET)) == dloc_ref[rp:rp + 1, :]).astype(jnp.bfloat16)
        m = lax.dot_general(at, bt, (((1,), (1,)), ((), ())),
                            preferred_element_type=jnp.float32)
        bm_ref[0, r] = m
    bm_ref[0, NHASH] = jnp.zeros((BUCKET, BUCKET), jnp.float32)


def _dst_kernel(buckets):
    return pl.pallas_call(
        _dst_body,
        grid=(HEADS,),
        in_specs=[pl.BlockSpec((1, NHASH, T), lambda h: (h, 0, 0))],
        out_specs=[
            pl.BlockSpec((1, NHASH, T), lambda h: (h, 0, 0)),
            pl.BlockSpec((1, NHASH + 1, BUCKET, BUCKET), lambda h: (h, 0, 0, 0)),
        ],
        out_shape=[
            jax.ShapeDtypeStruct((HEADS, NHASH, T), jnp.int32),
            jax.ShapeDtypeStruct((HEADS, NHASH + 1, BUCKET, BUCKET), jnp.float32),
        ],
        scratch_shapes=[pltpu.VMEM((NHASH, T), jnp.int32),
                        pltpu.VMEM((NB, T), jnp.bfloat16)],
    )(buckets)


def _sc_scatter(qkv_flat, dstg4):
    mesh = plsc.VectorSubcoreMesh(core_axis_name="c", subcore_axis_name="s")

    NG = 16

    @functools.partial(
        pl.kernel,
        out_type=jax.ShapeDtypeStruct((HEADS * NR, 2 * DH), jnp.float32),
        mesh=mesh,
        scratch_types=[
            pltpu.VMEM((T // 128, 128), jnp.int32),
            pltpu.VMEM((256, 2 * DH), jnp.float32),
            pltpu.VMEM((256, 2 * DH), jnp.float32),
            pltpu.SemaphoreType.DMA,
            pltpu.SemaphoreType.DMA,
        ],
    )
    def k(qkv_hbm, dstg_hbm, sqkv_out, gidx, ra, rb, sa, sb):
        wid = lax.axis_index("s") * 2 + lax.axis_index("c")
        bufs = (ra, rb)
        sems = (sa, sb)
        for j in range(NTASK // 32):
            task = wid * (NTASK // 32) + j
            h = task // NHASH
            r = task % NHASH
            pltpu.sync_copy(dstg_hbm.at[h, r], gidx)
            for i in range(NG):
                b = bufs[i % 2]
                sem = sems[i % 2]
                if i >= 2:
                    for k2 in range(2):
                        pltpu.make_async_copy(
                            qkv_hbm.at[pl.ds(0, 128)],
                            b.at[pl.ds(k2 * 128, 128)], sem).wait()
                pltpu.sync_copy(qkv_hbm.at[pl.ds(h * T + i * 256, 256)], b)
                for k2 in range(2):
                    pltpu.async_copy(b.at[pl.ds(k2 * 128, 128)],
                                     sqkv_out.at[gidx.at[i * 2 + k2]], sem)
            for i in (NG - 2, NG - 1):
                b = bufs[i % 2]
                sem = sems[i % 2]
                for k2 in range(2):
                    pltpu.make_async_copy(
                        qkv_hbm.at[pl.ds(0, 128)],
                        b.at[pl.ds(k2 * 128, 128)], sem).wait()

    return k(qkv_flat, dstg4)


_W = 4


def _attn_body(qkv_ref, qkvp_ref, bm_ref, so_ref):
    B = BUCKET
    M = _W * B
    N = (_W + 1) * B
    ri = lax.broadcasted_iota(jnp.int32, (M, N), 0)
    ci = lax.broadcasted_iota(jnp.int32, (M, N), 1)
    rb = (ri // B) * B
    keep = ((ci >= rb) & (ci < rb + 2 * B)).astype(jnp.float32)
    eye2 = (ci == ri + B).astype(jnp.float32) * 1e5
    bm = bm_ref[0, 0]
    bmp = jnp.concatenate(
        [jnp.concatenate([bm * 1e5, jnp.zeros((B, N - B), jnp.float32)],
                         axis=1),
         jnp.zeros((M - B, N), jnp.float32)], axis=0)
    scale = DH ** -0.5

    def norm(kk):
        return (kk * lax.rsqrt(
            jnp.sum(kk * kk, axis=1, keepdims=True) + 1e-6)
                ).astype(jnp.bfloat16)

    for w in range(CB // _W):
        blks = [qkv_ref[0, w * _W + k] for k in range(_W)]
        prev = qkv_ref[0, w * _W - 1] if w > 0 else qkvp_ref[0, 0]
        q4 = jnp.concatenate([b[:, 0:DH] for b in blks],
                             axis=0).astype(jnp.bfloat16)
        kn5 = jnp.concatenate(
            [norm(prev[:, 0:DH])] + [norm(b[:, 0:DH]) for b in blks],
            axis=0)
        v5 = jnp.concatenate(
            [prev[:, DH:2 * DH]] + [b[:, DH:2 * DH] for b in blks],
            axis=0).astype(jnp.bfloat16)
        dots = lax.dot_general(q4, kn5, (((1,), (1,)), ((), ())),
                               preferred_element_type=jnp.float32) * scale
        dots = dots - eye2
        if w == 0:
            dots = dots - bmp
        mx = jnp.max(dots, axis=1, keepdims=True)
        e = jnp.exp(dots - mx) * keep
        s = jnp.sum(e, axis=1, keepdims=True)
        lse = mx + jnp.log(s)
        p = (e * (1.0 / s)).astype(jnp.bfloat16)
        o = jnp.dot(p, v5, preferred_element_type=jnp.float32)
        so_ref[0, pl.ds(w * M, M), 0:DH] = o
        so_ref[0, pl.ds(w * M, M), DH:OW] = jnp.broadcast_to(
            lse, (M, OW - DH))


def _attention(sqkv4, bmask):
    return pl.pallas_call(
        _attn_body,
        grid=(HEADS, NCH // CB),
        in_specs=[
            pl.BlockSpec((1, CB, BUCKET, 2 * DH), lambda h, c: (h, c, 0, 0)),
            pl.BlockSpec((1, 1, BUCKET, 2 * DH),
                         lambda h, c: (h, (c * CB - 1) % NCH, 0, 0)),
            pl.BlockSpec((1, 1, BUCKET, BUCKET),
                         lambda h, c: (h,
                                       jnp.where(c % (NB // CB) == 0,
                                                 c // (NB // CB), NHASH),
                                       0, 0)),
        ],
        out_specs=pl.BlockSpec((1, CB * BUCKET, OW), lambda h, c: (h, c, 0)),
        out_shape=jax.ShapeDtypeStruct((HEADS, NR, OW), jnp.float32),
    )(sqkv4, sqkv4, bmask)


def _sc_unsort(so_flat, dstg4):
    mesh = plsc.VectorSubcoreMesh(core_axis_name="c", subcore_axis_name="s")

    NG = 16

    @functools.partial(
        pl.kernel,
        out_type=jax.ShapeDtypeStruct((HEADS * NR, OW), jnp.float32),
        mesh=mesh,
        scratch_types=[
            pltpu.VMEM((T // 128, 128), jnp.int32),
            pltpu.VMEM((256, OW), jnp.float32),
            pltpu.VMEM((256, OW), jnp.float32),
            pltpu.SemaphoreType.DMA,
            pltpu.SemaphoreType.DMA,
        ],
    )
    def k(so_hbm, dstg_hbm, og_out, gidx, ra, rb, sa, sb):
        wid = lax.axis_index("s") * 2 + lax.axis_index("c")
        bufs = (ra, rb)
        sems = (sa, sb)
        for j in range(NTASK // 32):
            task = wid * (NTASK // 32) + j
            h = task // NHASH
            r = task % NHASH
            pltpu.sync_copy(dstg_hbm.at[h, r], gidx)
            base = h * NR + r * T

            def drain_store(i):
                b = bufs[i % 2]
                sem = sems[i % 2]
                for k2 in range(2):
                    pltpu.make_async_copy(
                        so_hbm.at[pl.ds(0, 128)],
                        b.at[pl.ds(k2 * 128, 128)], sem).wait()
                pltpu.sync_copy(b, og_out.at[pl.ds(base + i * 256, 256)])

            for i in range(NG):
                b = bufs[i % 2]
                sem = sems[i % 2]
                for k2 in range(2):
                    pltpu.async_copy(so_hbm.at[gidx.at[i * 2 + k2]],
                                     b.at[pl.ds(k2 * 128, 128)], sem)
                if i >= 1:
                    drain_store(i - 1)
            drain_store(NG - 1)

    return k(so_flat, dstg4)


def _combine_body(og_ref, wo_ref, bo_ref, out_ref):
    h = pl.program_id(1)
    l3 = og_ref[0, :, :, DH:DH + 1]
    mx = jnp.max(l3, axis=0, keepdims=True)
    pe = jnp.exp(l3 - mx)
    ps = jnp.sum(pe, axis=0, keepdims=True)
    pr = pe / ps
    attn = jnp.sum(og_ref[0, :, :, 0:DH] * pr, axis=0)
    contrib = jnp.dot(attn.astype(jnp.bfloat16), wo_ref[0],
                      preferred_element_type=jnp.float32)

    @pl.when(h == 0)
    def _():
        out_ref[...] = contrib + bo_ref[...]

    @pl.when(h != 0)
    def _():
        out_ref[...] += contrib


_TT = 1024


def _combine(og4, Wo3, bo2):
    return pl.pallas_call(
        _combine_body,
        grid=(T // _TT, HEADS),
        in_specs=[
            pl.BlockSpec((1, NHASH, _TT, OW), lambda tt, h: (h, 0, tt, 0)),
            pl.BlockSpec((1, DH, EMB), lambda tt, h: (h, 0, 0)),
            pl.BlockSpec((1, EMB), lambda tt, h: (0, 0)),
        ],
        out_specs=pl.BlockSpec((_TT, EMB), lambda tt, h: (tt, 0)),
        out_shape=jax.ShapeDtypeStruct((T, EMB), jnp.float32),
    )(og4, Wo3, bo2)


def kernel(x, Wqk, Wv, Wo, bo, rotations):
    b, t, e = x.shape
    x2 = x.reshape(t, e)
    rotT = rotations.reshape(DH, NHASH * (NB // 2)).T
    Wqk3 = Wqk.reshape(EMB, HEADS, DH).transpose(1, 0, 2)
    Wv3 = Wv.reshape(EMB, HEADS, DH).transpose(1, 0, 2)
    qkv, buckets = _proj_hash(x2, Wqk3, Wv3, rotT)
    dstg, bmask = _dst_kernel(buckets)
    sqkv = _sc_scatter(qkv.reshape(HEADS * T, 2 * DH),
                       dstg.reshape(HEADS, NHASH, T // 128, 128))
    so = _attention(sqkv.reshape(HEADS, NCH, BUCKET, 2 * DH), bmask)
    og = _sc_unsort(so.reshape(HEADS * NR, OW),
                    dstg.reshape(HEADS, NHASH, T // 128, 128))
    out = _combine(og.reshape(HEADS, NHASH, T, OW),
                   Wo.reshape(HEADS, DH, EMB).astype(jnp.bfloat16),
                   bo.reshape(1, EMB))
    return out.reshape(b, t, e)


# speedup vs baseline: 1.1006x; 1.1006x over previous
import functools
import jax
import jax.numpy as jnp
from jax import lax
from jax.experimental import pallas as pl
from jax.experimental.pallas import tpu as pltpu
from jax.experimental.pallas import tpu_sc as plsc

EMB = 768
HEADS = 12
BUCKET = 64
NHASH = 8
T = 4096
DH = EMB // HEADS
NB = T // BUCKET
NCH = NHASH * T // BUCKET
CB = 16
NR = NHASH * T
NTASK = HEADS * NHASH
OW = 128


def _proj_hash_body(x_ref, wqk_ref, wv_ref, rot_ref, qkv_ref, b_ref):
    qk = jnp.dot(x_ref[...], wqk_ref[0], preferred_element_type=jnp.float32)
    v = jnp.dot(x_ref[...], wv_ref[0], preferred_element_type=jnp.float32)
    qkv_ref[0, :, 0:DH] = qk
    qkv_ref[0, :, DH:2 * DH] = v
    rT = lax.dot_general(rot_ref[...], qk, (((1,), (1,)), ((), ())),
                         preferred_element_type=jnp.float32)
    half = NB // 2
    sub_iota = lax.broadcasted_iota(jnp.int32, (NB, T), 0)
    for r in range(NHASH):
        rr = rT[half * r:half * (r + 1), :]
        full = jnp.concatenate([rr, -rr], axis=0)
        mx = jnp.max(full, axis=0, keepdims=True)
        idx = jnp.min(jnp.where(full == mx, sub_iota, NB), axis=0, keepdims=True)
        b_ref[0, r:r + 1, :] = idx


def _proj_hash(x2, Wqk3, Wv3, rotT):
    return pl.pallas_call(
        _proj_hash_body,
        grid=(HEADS,),
        in_specs=[
            pl.BlockSpec((T, EMB), lambda h: (0, 0)),
            pl.BlockSpec((1, EMB, DH), lambda h: (h, 0, 0)),
            pl.BlockSpec((1, EMB, DH), lambda h: (h, 0, 0)),
            pl.BlockSpec((NHASH * (NB // 2), DH), lambda h: (0, 0)),
        ],
        out_specs=[
            pl.BlockSpec((1, T, 2 * DH), lambda h: (h, 0, 0)),
            pl.BlockSpec((1, NHASH, T), lambda h: (h, 0, 0)),
        ],
        out_shape=[
            jax.ShapeDtypeStruct((HEADS, T, 2 * DH), jnp.float32),
            jax.ShapeDtypeStruct((HEADS, NHASH, T), jnp.int32),
        ],
    )(x2, Wqk3, Wv3, rotT)


def _dst_body(b_ref, dst_ref, bm_ref, dloc_ref, ot_ref):
    h = pl.program_id(0)
    C = 128
    NCHK = T // C
    su = (lax.broadcasted_iota(jnp.int32, (C, C), 0)
          < lax.broadcasted_iota(jnp.int32, (C, C), 1)).astype(jnp.bfloat16)
    sl = (lax.broadcasted_iota(jnp.int32, (NB, NB), 1)
          < lax.broadcasted_iota(jnp.int32, (NB, NB), 0)).astype(jnp.float32)
    iota_col = lax.broadcasted_iota(jnp.int32, (NB, C), 0)

    for r in range(NHASH):
        hist = jnp.zeros((NB, 1), jnp.float32)
        for i in range(NCHK):
            b_row = b_ref[0, r:r + 1, pl.ds(i * C, C)]
            otb = (iota_col == b_row).astype(jnp.bfloat16)
            ot_ref[:, pl.ds(i * C, C)] = otb
            hist = hist + jnp.sum(otb, axis=1, keepdims=True
                                  ).astype(jnp.float32)
        offs = jnp.dot(sl, hist, preferred_element_type=jnp.float32)
        goff = (h * NHASH + r) * T

        base = jnp.zeros((NB, 1), jnp.float32)
        for i in range(NCHK):
            otb = ot_ref[:, pl.ds(i * C, C)]
            rank = jnp.dot(otb, su, preferred_element_type=jnp.float32)
            val = (rank + base + offs) * otb.astype(jnp.float32)
            dstv = jnp.sum(val, axis=0, keepdims=True)
            dloc_ref[r:r + 1, pl.ds(i * C, C)] = dstv.astype(jnp.int32)
            dst_ref[0, r:r + 1, pl.ds(i * C, C)] = dstv.astype(jnp.int32) + goff
            base = base + jnp.sum(otb, axis=1, keepdims=True
                                  ).astype(jnp.float32)

    iota_bk = lax.broadcasted_iota(jnp.int32, (BUCKET, T), 0)
    for r in range(NHASH):
        rp = (r - 1) % NHASH
        at = (iota_bk == dloc_ref[r:r + 1, :]).astype(jnp.bfloat16)
        bt = ((iota_bk + (T - BUCKET)) == dloc_ref[rp:rp + 1, :]).astype(jnp.bfloat16)
        m = lax.dot_general(at, bt, (((1,), (1,)), ((), ())),
                            preferred_element_type=jnp.float32)
        bm_ref[0, r] = m
    bm_ref[0, NHASH] = jnp.zeros((BUCKET, BUCKET), jnp.float32)


def _dst_kernel(buckets):
    return pl.pallas_call(
        _dst_body,
        grid=(HEADS,),
        in_specs=[pl.BlockSpec((1, NHASH, T), lambda h: (h, 0, 0))],
        out_specs=[
            pl.BlockSpec((1, NHASH, T), lambda h: (h, 0, 0)),
            pl.BlockSpec((1, NHASH + 1, BUCKET, BUCKET), lambda h: (h, 0, 0, 0)),
        ],
        out_shape=[
            jax.ShapeDtypeStruct((HEADS, NHASH, T), jnp.int32),
            jax.ShapeDtypeStruct((HEADS, NHASH + 1, BUCKET, BUCKET), jnp.float32),
        ],
        scratch_shapes=[pltpu.VMEM((NHASH, T), jnp.int32),
                        pltpu.VMEM((NB, T), jnp.bfloat16)],
    )(buckets)


def _sc_scatter(qkv_flat, dstg4):
    mesh = plsc.VectorSubcoreMesh(core_axis_name="c", subcore_axis_name="s")

    NG = 16

    @functools.partial(
        pl.kernel,
        out_type=jax.ShapeDtypeStruct((HEADS * NR, 2 * DH), jnp.float32),
        mesh=mesh,
        scratch_types=[
            pltpu.VMEM((T // 128, 128), jnp.int32),
            pltpu.VMEM((256, 2 * DH), jnp.float32),
            pltpu.VMEM((256, 2 * DH), jnp.float32),
            pltpu.SemaphoreType.DMA,
            pltpu.SemaphoreType.DMA,
        ],
    )
    def k(qkv_hbm, dstg_hbm, sqkv_out, gidx, ra, rb, sa, sb):
        wid = lax.axis_index("s") * 2 + lax.axis_index("c")
        bufs = (ra, rb)
        sems = (sa, sb)
        for j in range(NTASK // 32):
            task = wid * (NTASK // 32) + j
            h = task // NHASH
            r = task % NHASH
            pltpu.sync_copy(dstg_hbm.at[h, r], gidx)
            for i in range(NG):
                b = bufs[i % 2]
                sem = sems[i % 2]
                if i >= 2:
                    for k2 in range(2):
                        pltpu.make_async_copy(
                            qkv_hbm.at[pl.ds(0, 128)],
                            b.at[pl.ds(k2 * 128, 128)], sem).wait()
                pltpu.sync_copy(qkv_hbm.at[pl.ds(h * T + i * 256, 256)], b)
                for k2 in range(2):
                    pltpu.async_copy(b.at[pl.ds(k2 * 128, 128)],
                                     sqkv_out.at[gidx.at[i * 2 + k2]], sem)
            for i in (NG - 2, NG - 1):
                b = bufs[i % 2]
                sem = sems[i % 2]
                for k2 in range(2):
                    pltpu.make_async_copy(
                        qkv_hbm.at[pl.ds(0, 128)],
                        b.at[pl.ds(k2 * 128, 128)], sem).wait()

    return k(qkv_flat, dstg4)


def _attn_body(qkv_ref, qkvp_ref, bm_ref, so_ref):
    B = BUCKET
    ri = lax.broadcasted_iota(jnp.int32, (2 * B, 4 * B), 0)
    ci = lax.broadcasted_iota(jnp.int32, (2 * B, 4 * B), 1)
    eye2 = (((ri == ci) & (ci < B))
            | ((ri == ci - B) & (ci >= 2 * B) & (ci < 3 * B))
            ).astype(jnp.float32) * 1e5
    keep = jnp.where(ri < B, (ci < 2 * B).astype(jnp.float32),
                     (ci >= 2 * B).astype(jnp.float32))
    bm = bm_ref[0, 0]
    zb = jnp.zeros((B, B), jnp.float32)
    bmp = jnp.concatenate(
        [jnp.concatenate([zb, bm * 1e5, zb, zb], axis=1),
         jnp.zeros((B, 4 * B), jnp.float32)], axis=0)
    scale = DH ** -0.5

    def norm(kk):
        return (kk * lax.rsqrt(
            jnp.sum(kk * kk, axis=1, keepdims=True) + 1e-6)
                ).astype(jnp.bfloat16)

    for jj in range(CB // 2):
        j = 2 * jj
        blka = qkv_ref[0, j]
        blkb = qkv_ref[0, j + 1]
        pblk = qkv_ref[0, j - 1] if j > 0 else qkvp_ref[0, 0]
        qa = blka[:, 0:DH]
        qb = blkb[:, 0:DH]
        q2 = jnp.concatenate([qa, qb], axis=0).astype(jnp.bfloat16)
        na = norm(qa)
        kn2 = jnp.concatenate([na, norm(pblk[:, 0:DH]), norm(qb), na], axis=0)
        v2 = jnp.concatenate(
            [blka[:, DH:2 * DH], pblk[:, DH:2 * DH],
             blkb[:, DH:2 * DH], blka[:, DH:2 * DH]],
            axis=0).astype(jnp.bfloat16)
        dots = lax.dot_general(q2, kn2, (((1,), (1,)), ((), ())),
                               preferred_element_type=jnp.float32) * scale
        dots = dots - eye2
        if jj == 0:
            dots = dots - bmp
        mx = jnp.max(dots, axis=1, keepdims=True)
        e = jnp.exp(dots - mx) * keep
        s = jnp.sum(e, axis=1, keepdims=True)
        lse = mx + jnp.log(s)
        p = (e * (1.0 / s)).astype(jnp.bfloat16)
        o = jnp.dot(p, v2, preferred_element_type=jnp.float32)
        so_ref[0, pl.ds(j * B, 2 * B), 0:DH] = o
        so_ref[0, pl.ds(j * B, 2 * B), DH:OW] = jnp.broadcast_to(
            lse, (2 * B, OW - DH))


def _attention(sqkv4, bmask):
    return pl.pallas_call(
        _attn_body,
        grid=(HEADS, NCH // CB),
        in_specs=[
            pl.BlockSpec((1, CB, BUCKET, 2 * DH), lambda h, c: (h, c, 0, 0)),
            pl.BlockSpec((1, 1, BUCKET, 2 * DH),
                         lambda h, c: (h, (c * CB - 1) % NCH, 0, 0)),
            pl.BlockSpec((1, 1, BUCKET, BUCKET),
                         lambda h, c: (h,
                                       jnp.where(c % (NB // CB) == 0,
                                                 c // (NB // CB), NHASH),
                                       0, 0)),
        ],
        out_specs=pl.BlockSpec((1, CB * BUCKET, OW), lambda h, c: (h, c, 0)),
        out_shape=jax.ShapeDtypeStruct((HEADS, NR, OW), jnp.float32),
    )(sqkv4, sqkv4, bmask)


def _sc_unsort(so_flat, dstg4):
    mesh = plsc.VectorSubcoreMesh(core_axis_name="c", subcore_axis_name="s")

    NG = 16

    @functools.partial(
        pl.kernel,
        out_type=jax.ShapeDtypeStruct((HEADS * NR, OW), jnp.float32),
        mesh=mesh,
        scratch_types=[
            pltpu.VMEM((T // 128, 128), jnp.int32),
            pltpu.VMEM((256, OW), jnp.float32),
            pltpu.VMEM((256, OW), jnp.float32),
            pltpu.SemaphoreType.DMA,
            pltpu.SemaphoreType.DMA,
        ],
    )
    def k(so_hbm, dstg_hbm, og_out, gidx, ra, rb, sa, sb):
        wid = lax.axis_index("s") * 2 + lax.axis_index("c")
        bufs = (ra, rb)
        sems = (sa, sb)
        for j in range(NTASK // 32):
            task = wid * (NTASK // 32) + j
            h = task // NHASH
            r = task % NHASH
            pltpu.sync_copy(dstg_hbm.at[h, r], gidx)
            base = h * NR + r * T

            def drain_store(i):
                b = bufs[i % 2]
                sem = sems[i % 2]
                for k2 in range(2):
                    pltpu.make_async_copy(
                        so_hbm.at[pl.ds(0, 128)],
                        b.at[pl.ds(k2 * 128, 128)], sem).wait()
                pltpu.sync_copy(b, og_out.at[pl.ds(base + i * 256, 256)])

            for i in range(NG):
                b = bufs[i % 2]
                sem = sems[i % 2]
                for k2 in range(2):
                    pltpu.async_copy(so_hbm.at[gidx.at[i * 2 + k2]],
                                     b.at[pl.ds(k2 * 128, 128)], sem)
                if i >= 1:
                    drain_store(i - 1)
            drain_store(NG - 1)

    return k(so_flat, dstg4)


def _combine_body(og_ref, wo_ref, bo_ref, out_ref):
    h = pl.program_id(1)
    l3 = og_ref[0, :, :, DH:DH + 1]
    mx = jnp.max(l3, axis=0, keepdims=True)
    pe = jnp.exp(l3 - mx)
    ps = jnp.sum(pe, axis=0, keepdims=True)
    pr = pe / ps
    attn = jnp.sum(og_ref[0, :, :, 0:DH] * pr, axis=0)
    contrib = jnp.dot(attn.astype(jnp.bfloat16), wo_ref[0],
                      preferred_element_type=jnp.float32)

    @pl.when(h == 0)
    def _():
        out_ref[...] = contrib + bo_ref[...]

    @pl.when(h != 0)
    def _():
        out_ref[...] += contrib


_TT = 1024


def _combine(og4, Wo3, bo2):
    return pl.pallas_call(
        _combine_body,
        grid=(T // _TT, HEADS),
        in_specs=[
            pl.BlockSpec((1, NHASH, _TT, OW), lambda tt, h: (h, 0, tt, 0)),
            pl.BlockSpec((1, DH, EMB), lambda tt, h: (h, 0, 0)),
            pl.BlockSpec((1, EMB), lambda tt, h: (0, 0)),
        ],
        out_specs=pl.BlockSpec((_TT, EMB), lambda tt, h: (tt, 0)),
        out_shape=jax.ShapeDtypeStruct((T, EMB), jnp.float32),
    )(og4, Wo3, bo2)


def kernel(x, Wqk, Wv, Wo, bo, rotations):
    b, t, e = x.shape
    x2 = x.reshape(t, e)
    rotT = rotations.reshape(DH, NHASH * (NB // 2)).T
    Wqk3 = Wqk.reshape(EMB, HEADS, DH).transpose(1, 0, 2)
    Wv3 = Wv.reshape(EMB, HEADS, DH).transpose(1, 0, 2)
    qkv, buckets = _proj_hash(x2, Wqk3, Wv3, rotT)
    dstg, bmask = _dst_kernel(buckets)
    sqkv = _sc_scatter(qkv.reshape(HEADS * T, 2 * DH),
                       dstg.reshape(HEADS, NHASH, T // 128, 128))
    so = _attention(sqkv.reshape(HEADS, NCH, BUCKET, 2 * DH), bmask)
    og = _sc_unsort(so.reshape(HEADS * NR, OW),
                    dstg.reshape(HEADS, NHASH, T // 128, 128))
    out = _combine(og.reshape(HEADS, NHASH, T, OW),
                   Wo.reshape(HEADS, DH, EMB).astype(jnp.bfloat16),
                   bo.reshape(1, EMB))
    return out.reshape(b, t, e)
